# Initial kernel scaffold; baseline (speedup 1.0000x reference)
import functools

import jax
import jax.numpy as jnp
from jax import lax
from jax.experimental import pallas as pl
from jax.experimental.pallas import tpu as pltpu
from jax.experimental.pallas import tpu_sc as plsc

N = 10000
E = 320000
NC, NS = 2, 16
NW = NC * NS
EPW = E // NW
C = 80
NCH = EPW // C
NP = 10240
RPT = NP // NS


def _sc_layer(D):
    mesh = plsc.VectorSubcoreMesh(core_axis_name="c", subcore_axis_name="s")

    @functools.partial(
        pl.kernel,
        out_type=[
            jax.ShapeDtypeStruct((NC, NP), jnp.float32),
            jax.ShapeDtypeStruct((NC, NP, D), jnp.float32),
        ],
        mesh=mesh,
        scratch_types=[
            pltpu.VMEM((NCH, C), jnp.int32),
            pltpu.VMEM((NCH, C), jnp.int32),
            pltpu.VMEM((NP,), jnp.float32),
            pltpu.VMEM((NP,), jnp.float32),
            pltpu.VMEM((EPW,), jnp.float32),
            pltpu.VMEM((C, D), jnp.float32),
            pltpu.VMEM((RPT,), jnp.float32),
            pltpu.VMEM_SHARED((NP,), jnp.float32),
            pltpu.VMEM_SHARED((NP, D), jnp.float32),
        ],
    )
    def k(src_hbm, dst_hbm, s_hbm, d_hbm, h_hbm, den_out, out_out,
          srcv, dstv, sv, dv, exv, rows, zrow, dacc, oacc):
        cid = lax.axis_index("c")
        sid = lax.axis_index("s")
        wid = sid * NC + cid

        zero = jnp.zeros((16,), jnp.float32)

        @pl.loop(0, RPT // 16)
        def _(i):
            zrow[pl.ds(i * 16, 16)] = zero

        @pl.loop(0, C)
        def _(r):
            for t in range(D // 16):
                rows[r, pl.ds(t * 16, 16)] = zero

        pltpu.sync_copy(zrow, dacc.at[pl.ds(sid * RPT, RPT)])
        for t in range(RPT // C):
            pltpu.sync_copy(rows, oacc.at[pl.ds(sid * RPT + t * C, C)])

        pltpu.sync_copy(src_hbm.at[wid], srcv)
        pltpu.sync_copy(dst_hbm.at[wid], dstv)
        pltpu.sync_copy(s_hbm, sv)
        pltpu.sync_copy(d_hbm, dv)

        @pl.loop(0, NCH)
        def _(j):
            @pl.loop(0, C // 16)
            def _(t):
                si = srcv[j, pl.ds(t * 16, 16)]
                di = dstv[j, pl.ds(t * 16, 16)]
                e = plsc.load_gather(sv, [si]) + plsc.load_gather(dv, [di])
                e = jnp.where(e >= 0.0, e, 0.2 * e)
                exv[pl.ds(j * C + t * 16, 16)] = jnp.exp(e)

        plsc.subcore_barrier()

        @pl.loop(0, NCH)
        def _(j):
            pltpu.sync_copy(exv.at[pl.ds(j * C, C)],
                            dacc.at[dstv.at[j]], add=True)
            pltpu.sync_copy(h_hbm.at[srcv.at[j]], rows)

            @pl.loop(0, C)
            def _(r):
                a = exv[j * C + r]
                for t in range(D // 16):
                    rows[r, pl.ds(t * 16, 16)] = rows[r, pl.ds(t * 16, 16)] * a

            pltpu.sync_copy(rows, oacc.at[dstv.at[j]], add=True)

        plsc.subcore_barrier()

        pltpu.sync_copy(dacc.at[pl.ds(sid * RPT, RPT)],
                        den_out.at[cid, pl.ds(sid * RPT, RPT)])
        pltpu.sync_copy(oacc.at[pl.ds(sid * RPT, RPT)],
                        out_out.at[cid, pl.ds(sid * RPT, RPT)])

    return k


_sc_layer1 = _sc_layer(64)
_sc_layer2 = _sc_layer(32)


def _tc_front(x, W1, A1):
    def body(x_ref, w_ref, a_ref, h_ref, sd_ref):
        h = jnp.dot(x_ref[...], w_ref[...], preferred_element_type=jnp.float32)
        h_ref[...] = h
        sd_ref[...] = jnp.dot(h, a_ref[...], preferred_element_type=jnp.float32)

    return pl.pallas_call(
        body,
        out_shape=[
            jax.ShapeDtypeStruct((NP, 64), jnp.float32),
            jax.ShapeDtypeStruct((NP, 2), jnp.float32),
        ],
    )(x, W1, A1)


def _tc_mid(op1, dp1, b1, W2, A2):
    def body(op_ref, dp_ref, b_ref, w_ref, a_ref, h_ref, sd_ref):
        acc = op_ref[0] + op_ref[1]
        den = dp_ref[0] + dp_ref[1]
        rden = 1.0 / (den + 1e-16)
        xx = acc * rden[:, None] + b_ref[...][None, :]
        xx = jnp.where(xx > 0.0, xx, jnp.expm1(xx))
        h2 = jnp.dot(xx, w_ref[...], preferred_element_type=jnp.float32)
        h_ref[...] = h2
        sd_ref[...] = jnp.dot(h2, a_ref[...], preferred_element_type=jnp.float32)

    return pl.pallas_call(
        body,
        out_shape=[
            jax.ShapeDtypeStruct((NP, 64), jnp.float32),
            jax.ShapeDtypeStruct((NP, 2), jnp.float32),
        ],
    )(op1, dp1, b1, W2, A2)


def _tc_back(op2, dp2, b2):
    def body(op_ref, dp_ref, b_ref, o_ref):
        acc = op_ref[0] + op_ref[1]
        den = dp_ref[0] + dp_ref[1]
        rden = 1.0 / (den + 1e-16)
        o_ref[...] = acc * rden[:, None] + b_ref[...][None, :]

    return pl.pallas_call(
        body,
        out_shape=jax.ShapeDtypeStruct((NP, 32), jnp.float32),
    )(op2, dp2, b2)


def kernel(x, edge_index, W1, a1_src, a1_dst, b1, W2, a2_src, a2_dst, b2):
    src = edge_index[0].reshape(NW, NCH, C)
    dst = edge_index[1].reshape(NW, NCH, C)
    A1 = jnp.stack([a1_src, a1_dst], axis=1)
    A2 = jnp.stack([a2_src, a2_dst], axis=1)
    xp = jnp.pad(x, ((0, NP - N), (0, 0)))

    h1, sd1 = _tc_front(xp, W1, A1)
    dp1, op1 = _sc_layer1(src, dst, sd1[:, 0], sd1[:, 1], h1)
    h2, sd2 = _tc_mid(op1, dp1, b1, W2, A2)
    dp2, op2 = _sc_layer2(src, dst, sd2[:, 0], sd2[:, 1], h2)
    out = _tc_back(op2, dp2, b2)
    return out[:N]


# speedup vs baseline: 24.1039x; 24.1039x over previous
import functools

import jax
import jax.numpy as jnp
from jax import lax
from jax.experimental import pallas as pl
from jax.experimental.pallas import tpu as pltpu
from jax.experimental.pallas import tpu_sc as plsc

N = 10000
E = 320000
NC, NS = 2, 16
NW = NC * NS
EPW = E // NW
C = 80
NCH = EPW // C
NP = 10240
RPT = NP // NS


def _sc_layer(D):
    mesh = plsc.VectorSubcoreMesh(core_axis_name="c", subcore_axis_name="s")

    @functools.partial(
        pl.kernel,
        out_type=[
            jax.ShapeDtypeStruct((NC, NP), jnp.float32),
            jax.ShapeDtypeStruct((NC, NP, D), jnp.float32),
        ],
        mesh=mesh,
        compiler_params=pltpu.CompilerParams(
            needs_layout_passes=False, use_tc_tiling_on_sc=False),
        scratch_types=[
            pltpu.VMEM((NCH, C), jnp.int32),
            pltpu.VMEM((NCH, C), jnp.int32),
            pltpu.VMEM((NP,), jnp.float32),
            pltpu.VMEM((NP,), jnp.float32),
            pltpu.VMEM((EPW,), jnp.float32),
            pltpu.VMEM((C, D), jnp.float32),
            pltpu.VMEM((RPT,), jnp.float32),
            pltpu.VMEM_SHARED((NP,), jnp.float32),
            pltpu.VMEM_SHARED((NP, D), jnp.float32),
        ],
    )
    def k(src_hbm, dst_hbm, s_hbm, d_hbm, h_hbm, den_out, out_out,
          srcv, dstv, sv, dv, exv, rows, zrow, dacc, oacc):
        cid = lax.axis_index("c")
        sid = lax.axis_index("s")
        wid = sid * NC + cid

        zero = jnp.zeros((16,), jnp.float32)

        @pl.loop(0, RPT // 16)
        def _(i):
            zrow[pl.ds(i * 16, 16)] = zero

        @pl.loop(0, C)
        def _(r):
            for t in range(D // 16):
                rows[r, pl.ds(t * 16, 16)] = zero

        pltpu.sync_copy(zrow, dacc.at[pl.ds(sid * RPT, RPT)])
        for t in range(RPT // C):
            pltpu.sync_copy(rows, oacc.at[pl.ds(sid * RPT + t * C, C)])

        pltpu.sync_copy(src_hbm.at[wid], srcv)
        pltpu.sync_copy(dst_hbm.at[wid], dstv)
        pltpu.sync_copy(s_hbm, sv)
        pltpu.sync_copy(d_hbm, dv)

        @pl.loop(0, NCH)
        def _(j):
            @pl.loop(0, C // 16)
            def _(t):
                si = srcv[j, pl.ds(t * 16, 16)]
                di = dstv[j, pl.ds(t * 16, 16)]
                e = plsc.load_gather(sv, [si]) + plsc.load_gather(dv, [di])
                e = jnp.where(e >= 0.0, e, 0.2 * e)
                exv[pl.ds(j * C + t * 16, 16)] = jnp.exp(e)

        plsc.subcore_barrier()

        @pl.loop(0, NCH)
        def _(j):
            pltpu.sync_copy(exv.at[pl.ds(j * C, C)],
                            dacc.at[dstv.at[j]], add=True)
            pltpu.sync_copy(h_hbm.at[srcv.at[j]], rows)

            @pl.loop(0, C)
            def _(r):
                idx = jnp.full((16,), j * C + r, jnp.int32)
                a = plsc.load_gather(exv, [idx])
                for t in range(D // 16):
                    rows[r, pl.ds(t * 16, 16)] = rows[r, pl.ds(t * 16, 16)] * a

            pltpu.sync_copy(rows, oacc.at[dstv.at[j]], add=True)

        plsc.subcore_barrier()

        pltpu.sync_copy(dacc.at[pl.ds(sid * RPT, RPT)],
                        den_out.at[cid, pl.ds(sid * RPT, RPT)])
        pltpu.sync_copy(oacc.at[pl.ds(sid * RPT, RPT)],
                        out_out.at[cid, pl.ds(sid * RPT, RPT)])

    return k


_sc_layer1 = _sc_layer(64)
_sc_layer2 = _sc_layer(32)


def _tc_front(x, W1, A1):
    def body(x_ref, w_ref, a_ref, h_ref, sd_ref):
        h = jnp.dot(x_ref[...], w_ref[...], preferred_element_type=jnp.float32)
        h_ref[...] = h
        sd_ref[...] = jnp.dot(h, a_ref[...], preferred_element_type=jnp.float32)

    return pl.pallas_call(
        body,
        out_shape=[
            jax.ShapeDtypeStruct((NP, 64), jnp.float32),
            jax.ShapeDtypeStruct((NP, 2), jnp.float32),
        ],
    )(x, W1, A1)


def _tc_mid(op1, dp1, b1, W2, A2):
    def body(op_ref, dp_ref, b_ref, w_ref, a_ref, h_ref, sd_ref):
        acc = op_ref[0] + op_ref[1]
        den = dp_ref[0] + dp_ref[1]
        rden = 1.0 / (den + 1e-16)
        xx = acc * rden[:, None] + b_ref[...][None, :]
        xx = jnp.where(xx > 0.0, xx, jnp.exp(xx) - 1.0)
        h2 = jnp.dot(xx, w_ref[...], preferred_element_type=jnp.float32)
        h_ref[...] = h2
        sd_ref[...] = jnp.dot(h2, a_ref[...], preferred_element_type=jnp.float32)

    return pl.pallas_call(
        body,
        out_shape=[
            jax.ShapeDtypeStruct((NP, 32), jnp.float32),
            jax.ShapeDtypeStruct((NP, 2), jnp.float32),
        ],
    )(op1, dp1, b1, W2, A2)


def _tc_back(op2, dp2, b2):
    def body(op_ref, dp_ref, b_ref, o_ref):
        acc = op_ref[0] + op_ref[1]
        den = dp_ref[0] + dp_ref[1]
        rden = 1.0 / (den + 1e-16)
        o_ref[...] = acc * rden[:, None] + b_ref[...][None, :]

    return pl.pallas_call(
        body,
        out_shape=jax.ShapeDtypeStruct((NP, 32), jnp.float32),
    )(op2, dp2, b2)


def kernel(x, edge_index, W1, a1_src, a1_dst, b1, W2, a2_src, a2_dst, b2):
    src = edge_index[0].reshape(NW, NCH, C)
    dst = edge_index[1].reshape(NW, NCH, C)
    A1 = jnp.stack([a1_src, a1_dst], axis=1)
    A2 = jnp.stack([a2_src, a2_dst], axis=1)
    xp = jnp.pad(x, ((0, NP - N), (0, 0)))

    h1, sd1 = _tc_front(xp, W1, A1)
    dp1, op1 = _sc_layer1(src, dst, sd1[:, 0], sd1[:, 1], h1)
    h2, sd2 = _tc_mid(op1, dp1, b1, W2, A2)
    dp2, op2 = _sc_layer2(src, dst, sd2[:, 0], sd2[:, 1], h2)
    out = _tc_back(op2, dp2, b2)
    return out[:N]


# speedup vs baseline: 26.1078x; 1.0831x over previous
import functools

import jax
import jax.numpy as jnp
from jax import lax
from jax.experimental import pallas as pl
from jax.experimental.pallas import tpu as pltpu
from jax.experimental.pallas import tpu_sc as plsc

N = 10000
E = 320000
NC, NS = 2, 16
NW = NC * NS
EPW = E // NW
CP = 128
EPWP = 10240
NCHP = EPWP // CP
NP = 10240
RPT = NP // NS
NBUF = 4


def _sc_layer(D):
    mesh = plsc.VectorSubcoreMesh(core_axis_name="c", subcore_axis_name="s")

    @functools.partial(
        pl.kernel,
        out_type=[
            jax.ShapeDtypeStruct((NC, NP), jnp.float32),
            jax.ShapeDtypeStruct((NC, NP, D), jnp.float32),
        ],
        mesh=mesh,
        compiler_params=pltpu.CompilerParams(
            needs_layout_passes=False, use_tc_tiling_on_sc=False),
        scratch_types=[
            pltpu.VMEM((NCHP, CP), jnp.int32),
            pltpu.VMEM((NCHP, CP), jnp.int32),
            pltpu.VMEM((NP,), jnp.float32),
            pltpu.VMEM((NP,), jnp.float32),
            pltpu.VMEM((EPWP,), jnp.float32),
            pltpu.VMEM((NBUF, CP, D), jnp.float32),
            pltpu.VMEM((RPT,), jnp.float32),
            pltpu.VMEM_SHARED((NP,), jnp.float32),
            pltpu.VMEM_SHARED((NP, D), jnp.float32),
            pltpu.SemaphoreType.DMA((NBUF,)),
            pltpu.SemaphoreType.DMA((NBUF,)),
            pltpu.SemaphoreType.DMA((NBUF,)),
        ],
    )
    def k(src_hbm, dst_hbm, s_hbm, d_hbm, h_hbm, den_out, out_out,
          srcv, dstv, sv, dv, exv, rows, zrow, dacc, oacc,
          gsem, ssem, dsem):
        cid = lax.axis_index("c")
        sid = lax.axis_index("s")
        wid = sid * NC + cid

        zero = jnp.zeros((16,), jnp.float32)

        pltpu.async_copy(src_hbm.at[wid], srcv, gsem.at[0])
        pltpu.async_copy(dst_hbm.at[wid], dstv, gsem.at[1])
        pltpu.async_copy(s_hbm, sv, gsem.at[2])
        pltpu.async_copy(d_hbm, dv, gsem.at[3])

        @pl.loop(0, RPT // 16, unroll=8)
        def _(i):
            zrow[pl.ds(i * 16, 16)] = zero

        @pl.loop(0, CP, unroll=4)
        def _(r):
            for t in range(D // 16):
                rows[0, r, pl.ds(t * 16, 16)] = zero

        pltpu.sync_copy(zrow, dacc.at[pl.ds(sid * RPT, RPT)])
        for t in range(RPT // CP):
            pltpu.sync_copy(rows.at[0], oacc.at[pl.ds(sid * RPT + t * CP, CP)])

        pltpu.make_async_copy(src_hbm.at[wid], srcv, gsem.at[0]).wait()
        pltpu.make_async_copy(dst_hbm.at[wid], dstv, gsem.at[1]).wait()
        pltpu.make_async_copy(s_hbm, sv, gsem.at[2]).wait()
        pltpu.make_async_copy(d_hbm, dv, gsem.at[3]).wait()

        @pl.loop(0, EPWP // 16, unroll=4)
        def _(v):
            row = v >> 3
            col = (v & 7) * 16
            si = srcv[row, pl.ds(col, 16)]
            di = dstv[row, pl.ds(col, 16)]
            e = plsc.load_gather(sv, [si]) + plsc.load_gather(dv, [di])
            e = jnp.where(e >= 0.0, e, 0.2 * e)
            exv[pl.ds(v * 16, 16)] = jnp.exp(e)

        plsc.subcore_barrier()

        for c in range(2):
            pltpu.async_copy(h_hbm.at[srcv.at[c]], rows.at[c], gsem.at[c])

        @pl.loop(0, NCHP, step=NBUF)
        def _(cbase):
            for b in range(NBUF):
                c = cbase + b
                bn = (b + 2) % NBUF

                @pl.when(c >= 2)
                def _():
                    pltpu.make_async_copy(
                        rows.at[bn], oacc.at[dstv.at[c - 2]],
                        ssem.at[bn]).wait()
                    pltpu.make_async_copy(
                        exv.at[pl.ds((c - 2) * CP, CP)],
                        dacc.at[dstv.at[c - 2]], dsem.at[bn]).wait()

                @pl.when(c < NCHP - 2)
                def _():
                    pltpu.async_copy(
                        h_hbm.at[srcv.at[c + 2]], rows.at[bn], gsem.at[bn])

                pltpu.make_async_copy(
                    h_hbm.at[srcv.at[c]], rows.at[b], gsem.at[b]).wait()

                @pl.loop(0, CP, unroll=2)
                def _(r):
                    idx = jnp.full((16,), c * CP + r, jnp.int32)
                    a = plsc.load_gather(exv, [idx])
                    for t in range(D // 16):
                        rows[b, r, pl.ds(t * 16, 16)] = (
                            rows[b, r, pl.ds(t * 16, 16)] * a)

                pltpu.async_copy(
                    rows.at[b], oacc.at[dstv.at[c]], ssem.at[b], add=True)
                pltpu.async_copy(
                    exv.at[pl.ds(c * CP, CP)], dacc.at[dstv.at[c]],
                    dsem.at[b], add=True)

        for c in (NCHP - 2, NCHP - 1):
            b = c % NBUF
            pltpu.make_async_copy(
                rows.at[b], oacc.at[dstv.at[c]], ssem.at[b]).wait()
            pltpu.make_async_copy(
                exv.at[pl.ds(c * CP, CP)], dacc.at[dstv.at[c]],
                dsem.at[b]).wait()

        plsc.subcore_barrier()

        pltpu.sync_copy(dacc.at[pl.ds(sid * RPT, RPT)],
                        den_out.at[cid, pl.ds(sid * RPT, RPT)])
        pltpu.sync_copy(oacc.at[pl.ds(sid * RPT, RPT)],
                        out_out.at[cid, pl.ds(sid * RPT, RPT)])

    return k


_sc_layer1 = _sc_layer(64)
_sc_layer2 = _sc_layer(32)


def _tc_front(x, W1, A1):
    def body(x_ref, w_ref, a_ref, h_ref, sd_ref):
        h = jnp.dot(x_ref[...], w_ref[...], preferred_element_type=jnp.float32)
        h_ref[...] = h
        sd_ref[...] = jnp.dot(h, a_ref[...], preferred_element_type=jnp.float32)

    return pl.pallas_call(
        body,
        out_shape=[
            jax.ShapeDtypeStruct((NP, 64), jnp.float32),
            jax.ShapeDtypeStruct((NP, 2), jnp.float32),
        ],
    )(x, W1, A1)


def _tc_mid(op1, dp1, b1, W2, A2):
    def body(op_ref, dp_ref, b_ref, w_ref, a_ref, h_ref, sd_ref):
        acc = op_ref[0] + op_ref[1]
        den = dp_ref[0] + dp_ref[1]
        rden = 1.0 / (den + 1e-16)
        xx = acc * rden[:, None] + b_ref[...][None, :]
        xx = jnp.where(xx > 0.0, xx, jnp.exp(xx) - 1.0)
        h2 = jnp.dot(xx, w_ref[...], preferred_element_type=jnp.float32)
        h_ref[...] = h2
        sd_ref[...] = jnp.dot(h2, a_ref[...], preferred_element_type=jnp.float32)

    return pl.pallas_call(
        body,
        out_shape=[
            jax.ShapeDtypeStruct((NP, 32), jnp.float32),
            jax.ShapeDtypeStruct((NP, 2), jnp.float32),
        ],
    )(op1, dp1, b1, W2, A2)


def _tc_back(op2, dp2, b2):
    def body(op_ref, dp_ref, b_ref, o_ref):
        acc = op_ref[0] + op_ref[1]
        den = dp_ref[0] + dp_ref[1]
        rden = 1.0 / (den + 1e-16)
        o_ref[...] = acc * rden[:, None] + b_ref[...][None, :]

    return pl.pallas_call(
        body,
        out_shape=jax.ShapeDtypeStruct((NP, 32), jnp.float32),
    )(op2, dp2, b2)


def kernel(x, edge_index, W1, a1_src, a1_dst, b1, W2, a2_src, a2_dst, b2):
    src = edge_index[0].reshape(NW, EPW)
    dst = edge_index[1].reshape(NW, EPW)
    src = jnp.pad(src, ((0, 0), (0, EPWP - EPW)), constant_values=N)
    dst = jnp.pad(dst, ((0, 0), (0, EPWP - EPW)), constant_values=N)
    src = src.reshape(NW, NCHP, CP)
    dst = dst.reshape(NW, NCHP, CP)
    A1 = jnp.stack([a1_src, a1_dst], axis=1)
    A2 = jnp.stack([a2_src, a2_dst], axis=1)
    xp = jnp.pad(x, ((0, NP - N), (0, 0)))

    h1, sd1 = _tc_front(xp, W1, A1)
    dp1, op1 = _sc_layer1(src, dst, sd1[:, 0], sd1[:, 1], h1)
    h2, sd2 = _tc_mid(op1, dp1, b1, W2, A2)
    dp2, op2 = _sc_layer2(src, dst, sd2[:, 0], sd2[:, 1], h2)
    out = _tc_back(op2, dp2, b2)
    return out[:N]


# speedup vs baseline: 30.3933x; 1.1641x over previous
import functools

import jax
import jax.numpy as jnp
from jax import lax
from jax.experimental import pallas as pl
from jax.experimental.pallas import tpu as pltpu
from jax.experimental.pallas import tpu_sc as plsc

N = 10000
E = 320000
NC, NS = 2, 16
NW = NC * NS
EPW = E // NW
CP = 128
EPWP = 10240
NCHP = EPWP // CP
NP = 10240
RPT = NP // NS
NBUF = 4


def _sc_layer(D):
    mesh = plsc.VectorSubcoreMesh(core_axis_name="c", subcore_axis_name="s")

    @functools.partial(
        pl.kernel,
        out_type=[
            jax.ShapeDtypeStruct((NC, NP), jnp.float32),
            jax.ShapeDtypeStruct((NC, NP, D), jnp.float32),
        ],
        mesh=mesh,
        compiler_params=pltpu.CompilerParams(
            needs_layout_passes=False, use_tc_tiling_on_sc=False),
        scratch_types=[
            pltpu.VMEM((NCHP, CP), jnp.int32),
            pltpu.VMEM((NCHP, CP), jnp.int32),
            pltpu.VMEM((NP,), jnp.float32),
            pltpu.VMEM((NP,), jnp.float32),
            pltpu.VMEM((EPWP,), jnp.float32),
            pltpu.VMEM((NBUF, CP, D), jnp.float32),
            pltpu.VMEM((RPT,), jnp.float32),
            pltpu.VMEM_SHARED((NP,), jnp.float32),
            pltpu.VMEM_SHARED((NP, D), jnp.float32),
            pltpu.SemaphoreType.DMA((NBUF,)),
            pltpu.SemaphoreType.DMA((NBUF,)),
            pltpu.SemaphoreType.DMA((NBUF,)),
        ],
    )
    def k(src_hbm, dst_hbm, s_hbm, d_hbm, h_hbm, den_out, out_out,
          srcv, dstv, sv, dv, exv, rows, zrow, dacc, oacc,
          gsem, ssem, dsem):
        cid = lax.axis_index("c")
        sid = lax.axis_index("s")
        wid = sid * NC + cid

        zero = jnp.zeros((16,), jnp.float32)

        pltpu.async_copy(src_hbm.at[wid], srcv, gsem.at[0])
        pltpu.async_copy(dst_hbm.at[wid], dstv, gsem.at[1])
        pltpu.async_copy(s_hbm, sv, gsem.at[2])
        pltpu.async_copy(d_hbm, dv, gsem.at[3])

        @pl.loop(0, RPT // 16, unroll=8)
        def _(i):
            zrow[pl.ds(i * 16, 16)] = zero

        @pl.loop(0, CP, unroll=4)
        def _(r):
            for t in range(D // 16):
                rows[0, r, pl.ds(t * 16, 16)] = zero

        pltpu.sync_copy(zrow, dacc.at[pl.ds(sid * RPT, RPT)])
        for t in range(RPT // CP):
            pltpu.sync_copy(rows.at[0], oacc.at[pl.ds(sid * RPT + t * CP, CP)])

        pltpu.make_async_copy(src_hbm.at[wid], srcv, gsem.at[0]).wait()
        pltpu.make_async_copy(dst_hbm.at[wid], dstv, gsem.at[1]).wait()
        pltpu.make_async_copy(s_hbm, sv, gsem.at[2]).wait()
        pltpu.make_async_copy(d_hbm, dv, gsem.at[3]).wait()

        @plsc.parallel_loop(0, EPWP // 16, unroll=8)
        def _(v):
            row = v >> 3
            col = (v & 7) * 16
            si = srcv[row, pl.ds(col, 16)]
            di = dstv[row, pl.ds(col, 16)]
            e = plsc.load_gather(sv, [si]) + plsc.load_gather(dv, [di])
            e = jnp.where(e >= 0.0, e, 0.2 * e)
            exv[pl.ds(v * 16, 16)] = jnp.exp(e)

        plsc.subcore_barrier()

        for c in range(2):
            pltpu.async_copy(h_hbm.at[srcv.at[c]], rows.at[c], gsem.at[c])

        @pl.loop(0, NCHP, step=NBUF)
        def _(cbase):
            for b in range(NBUF):
                c = cbase + b
                bn = (b + 2) % NBUF

                @pl.when(c >= 2)
                def _():
                    pltpu.make_async_copy(
                        rows.at[bn], oacc.at[dstv.at[c - 2]],
                        ssem.at[bn]).wait()
                    pltpu.make_async_copy(
                        exv.at[pl.ds((c - 2) * CP, CP)],
                        dacc.at[dstv.at[c - 2]], dsem.at[bn]).wait()

                @pl.when(c < NCHP - 2)
                def _():
                    pltpu.async_copy(
                        h_hbm.at[srcv.at[c + 2]], rows.at[bn], gsem.at[bn])

                pltpu.make_async_copy(
                    h_hbm.at[srcv.at[c]], rows.at[b], gsem.at[b]).wait()

                @plsc.parallel_loop(0, CP, unroll=8)
                def _(r):
                    idx = jnp.full((16,), c * CP + r, jnp.int32)
                    a = plsc.load_gather(exv, [idx])
                    for t in range(D // 16):
                        rows[b, r, pl.ds(t * 16, 16)] = (
                            rows[b, r, pl.ds(t * 16, 16)] * a)

                pltpu.async_copy(
                    rows.at[b], oacc.at[dstv.at[c]], ssem.at[b], add=True)
                pltpu.async_copy(
                    exv.at[pl.ds(c * CP, CP)], dacc.at[dstv.at[c]],
                    dsem.at[b], add=True)

        for c in (NCHP - 2, NCHP - 1):
            b = c % NBUF
            pltpu.make_async_copy(
                rows.at[b], oacc.at[dstv.at[c]], ssem.at[b]).wait()
            pltpu.make_async_copy(
                exv.at[pl.ds(c * CP, CP)], dacc.at[dstv.at[c]],
                dsem.at[b]).wait()

        plsc.subcore_barrier()

        pltpu.sync_copy(dacc.at[pl.ds(sid * RPT, RPT)],
                        den_out.at[cid, pl.ds(sid * RPT, RPT)])
        pltpu.sync_copy(oacc.at[pl.ds(sid * RPT, RPT)],
                        out_out.at[cid, pl.ds(sid * RPT, RPT)])

    return k


_sc_layer1 = _sc_layer(64)
_sc_layer2 = _sc_layer(32)


def _tc_front(x, W1, A1):
    def body(x_ref, w_ref, a_ref, h_ref, sd_ref):
        h = jnp.dot(x_ref[...], w_ref[...], preferred_element_type=jnp.float32)
        h_ref[...] = h
        sd_ref[...] = jnp.dot(h, a_ref[...], preferred_element_type=jnp.float32)

    return pl.pallas_call(
        body,
        out_shape=[
            jax.ShapeDtypeStruct((NP, 64), jnp.float32),
            jax.ShapeDtypeStruct((NP, 2), jnp.float32),
        ],
    )(x, W1, A1)


def _tc_mid(op1, dp1, b1, W2, A2):
    def body(op_ref, dp_ref, b_ref, w_ref, a_ref, h_ref, sd_ref):
        acc = op_ref[0] + op_ref[1]
        den = dp_ref[0] + dp_ref[1]
        rden = 1.0 / (den + 1e-16)
        xx = acc * rden[:, None] + b_ref[...][None, :]
        xx = jnp.where(xx > 0.0, xx, jnp.exp(xx) - 1.0)
        h2 = jnp.dot(xx, w_ref[...], preferred_element_type=jnp.float32)
        h_ref[...] = h2
        sd_ref[...] = jnp.dot(h2, a_ref[...], preferred_element_type=jnp.float32)

    return pl.pallas_call(
        body,
        out_shape=[
            jax.ShapeDtypeStruct((NP, 32), jnp.float32),
            jax.ShapeDtypeStruct((NP, 2), jnp.float32),
        ],
    )(op1, dp1, b1, W2, A2)


def _tc_back(op2, dp2, b2):
    def body(op_ref, dp_ref, b_ref, o_ref):
        acc = op_ref[0] + op_ref[1]
        den = dp_ref[0] + dp_ref[1]
        rden = 1.0 / (den + 1e-16)
        o_ref[...] = acc * rden[:, None] + b_ref[...][None, :]

    return pl.pallas_call(
        body,
        out_shape=jax.ShapeDtypeStruct((NP, 32), jnp.float32),
    )(op2, dp2, b2)


def kernel(x, edge_index, W1, a1_src, a1_dst, b1, W2, a2_src, a2_dst, b2):
    src = edge_index[0].reshape(NW, EPW)
    dst = edge_index[1].reshape(NW, EPW)
    src = jnp.pad(src, ((0, 0), (0, EPWP - EPW)), constant_values=N)
    dst = jnp.pad(dst, ((0, 0), (0, EPWP - EPW)), constant_values=N)
    src = src.reshape(NW, NCHP, CP)
    dst = dst.reshape(NW, NCHP, CP)
    A1 = jnp.stack([a1_src, a1_dst], axis=1)
    A2 = jnp.stack([a2_src, a2_dst], axis=1)
    xp = jnp.pad(x, ((0, NP - N), (0, 0)))

    h1, sd1 = _tc_front(xp, W1, A1)
    dp1, op1 = _sc_layer1(src, dst, sd1[:, 0], sd1[:, 1], h1)
    h2, sd2 = _tc_mid(op1, dp1, b1, W2, A2)
    dp2, op2 = _sc_layer2(src, dst, sd2[:, 0], sd2[:, 1], h2)
    out = _tc_back(op2, dp2, b2)
    return out[:N]


# speedup vs baseline: 34.8291x; 1.1459x over previous
import functools

import jax
import jax.numpy as jnp
from jax import lax
from jax.experimental import pallas as pl
from jax.experimental.pallas import tpu as pltpu
from jax.experimental.pallas import tpu_sc as plsc

N = 10000
E = 320000
NC, NS = 2, 16
NW = NC * NS
EPW = E // NW
CP = 128
EPWP = 10240
NCHP = EPWP // CP
NP = 10240
RPT = NP // NS
NBUF = 4


def _sc_layer(D, stage=True):
    mesh = plsc.VectorSubcoreMesh(core_axis_name="c", subcore_axis_name="s")

    @functools.partial(
        pl.kernel,
        out_type=[
            jax.ShapeDtypeStruct((NC, NP), jnp.float32),
            jax.ShapeDtypeStruct((NC, NP, D), jnp.float32),
        ],
        mesh=mesh,
        compiler_params=pltpu.CompilerParams(
            needs_layout_passes=False, use_tc_tiling_on_sc=False),
        scratch_types=[
            pltpu.VMEM((NCHP, CP), jnp.int32),
            pltpu.VMEM((NCHP, CP), jnp.int32),
            pltpu.VMEM((NP,), jnp.float32),
            pltpu.VMEM((NP,), jnp.float32),
            pltpu.VMEM((EPWP,), jnp.float32),
            pltpu.VMEM((NBUF, CP, D), jnp.float32),
            pltpu.VMEM((RPT,), jnp.float32),
            pltpu.VMEM_SHARED((NP,), jnp.float32),
            pltpu.VMEM_SHARED((NP, D), jnp.float32),
            pltpu.VMEM_SHARED((NP if stage else 1, D), jnp.float32),
            pltpu.SemaphoreType.DMA((NBUF,)),
            pltpu.SemaphoreType.DMA((NBUF,)),
            pltpu.SemaphoreType.DMA((NBUF,)),
            pltpu.SemaphoreType.DMA,
        ],
    )
    def k(src_hbm, dst_hbm, s_hbm, d_hbm, h_hbm, den_out, out_out,
          srcv, dstv, sv, dv, exv, rows, zrow, dacc, oacc, hbuf,
          gsem, ssem, dsem, hsem):
        cid = lax.axis_index("c")
        sid = lax.axis_index("s")
        wid = sid * NC + cid

        zero = jnp.zeros((16,), jnp.float32)

        pltpu.async_copy(src_hbm.at[wid], srcv, gsem.at[0])
        pltpu.async_copy(dst_hbm.at[wid], dstv, gsem.at[1])
        pltpu.async_copy(s_hbm, sv, gsem.at[2])
        pltpu.async_copy(d_hbm, dv, gsem.at[3])
        if stage:
            pltpu.async_copy(h_hbm.at[pl.ds(sid * RPT, RPT)],
                             hbuf.at[pl.ds(sid * RPT, RPT)], hsem)
        hsrc = hbuf if stage else h_hbm

        @pl.loop(0, RPT // 16, unroll=8)
        def _(i):
            zrow[pl.ds(i * 16, 16)] = zero

        @pl.loop(0, CP, unroll=4)
        def _(r):
            for t in range(D // 16):
                rows[0, r, pl.ds(t * 16, 16)] = zero

        pltpu.sync_copy(zrow, dacc.at[pl.ds(sid * RPT, RPT)])
        for t in range(RPT // CP):
            pltpu.sync_copy(rows.at[0], oacc.at[pl.ds(sid * RPT + t * CP, CP)])

        pltpu.make_async_copy(src_hbm.at[wid], srcv, gsem.at[0]).wait()
        pltpu.make_async_copy(dst_hbm.at[wid], dstv, gsem.at[1]).wait()
        pltpu.make_async_copy(s_hbm, sv, gsem.at[2]).wait()
        pltpu.make_async_copy(d_hbm, dv, gsem.at[3]).wait()

        @plsc.parallel_loop(0, EPWP // 16, unroll=8)
        def _(v):
            row = v >> 3
            col = (v & 7) * 16
            si = srcv[row, pl.ds(col, 16)]
            di = dstv[row, pl.ds(col, 16)]
            e = plsc.load_gather(sv, [si]) + plsc.load_gather(dv, [di])
            e = jnp.where(e >= 0.0, e, 0.2 * e)
            exv[pl.ds(v * 16, 16)] = jnp.exp(e)

        if stage:
            pltpu.make_async_copy(h_hbm.at[pl.ds(sid * RPT, RPT)],
                                  hbuf.at[pl.ds(sid * RPT, RPT)], hsem).wait()
        plsc.subcore_barrier()

        for c in range(2):
            pltpu.async_copy(hsrc.at[srcv.at[c]], rows.at[c], gsem.at[c])

        @pl.loop(0, NCHP, step=NBUF)
        def _(cbase):
            for b in range(NBUF):
                c = cbase + b
                bn = (b + 2) % NBUF

                @pl.when(c >= 2)
                def _():
                    pltpu.make_async_copy(
                        rows.at[bn], oacc.at[dstv.at[c - 2]],
                        ssem.at[bn]).wait()
                    pltpu.make_async_copy(
                        exv.at[pl.ds((c - 2) * CP, CP)],
                        dacc.at[dstv.at[c - 2]], dsem.at[bn]).wait()

                @pl.when(c < NCHP - 2)
                def _():
                    pltpu.async_copy(
                        hsrc.at[srcv.at[c + 2]], rows.at[bn], gsem.at[bn])

                pltpu.make_async_copy(
                    hsrc.at[srcv.at[c]], rows.at[b], gsem.at[b]).wait()

                @plsc.parallel_loop(0, CP, unroll=8)
                def _(r):
                    idx = jnp.full((16,), c * CP + r, jnp.int32)
                    a = plsc.load_gather(exv, [idx])
                    for t in range(D // 16):
                        rows[b, r, pl.ds(t * 16, 16)] = (
                            rows[b, r, pl.ds(t * 16, 16)] * a)

                pltpu.async_copy(
                    rows.at[b], oacc.at[dstv.at[c]], ssem.at[b], add=True)
                pltpu.async_copy(
                    exv.at[pl.ds(c * CP, CP)], dacc.at[dstv.at[c]],
                    dsem.at[b], add=True)

        for c in (NCHP - 2, NCHP - 1):
            b = c % NBUF
            pltpu.make_async_copy(
                rows.at[b], oacc.at[dstv.at[c]], ssem.at[b]).wait()
            pltpu.make_async_copy(
                exv.at[pl.ds(c * CP, CP)], dacc.at[dstv.at[c]],
                dsem.at[b]).wait()

        plsc.subcore_barrier()

        pltpu.sync_copy(dacc.at[pl.ds(sid * RPT, RPT)],
                        den_out.at[cid, pl.ds(sid * RPT, RPT)])
        pltpu.sync_copy(oacc.at[pl.ds(sid * RPT, RPT)],
                        out_out.at[cid, pl.ds(sid * RPT, RPT)])

    return k


_sc_layer1 = _sc_layer(64, stage=False)
_sc_layer2 = _sc_layer(32, stage=True)


def _tc_front(x, W1, A1):
    def body(x_ref, w_ref, a_ref, h_ref, sd_ref):
        h = jnp.dot(x_ref[...], w_ref[...], preferred_element_type=jnp.float32)
        h_ref[...] = h
        sd_ref[...] = jnp.dot(h, a_ref[...], preferred_element_type=jnp.float32)

    return pl.pallas_call(
        body,
        out_shape=[
            jax.ShapeDtypeStruct((NP, 64), jnp.float32),
            jax.ShapeDtypeStruct((NP, 2), jnp.float32),
        ],
    )(x, W1, A1)


def _tc_mid(op1, dp1, b1, W2, A2):
    def body(op_ref, dp_ref, b_ref, w_ref, a_ref, h_ref, sd_ref):
        acc = op_ref[0] + op_ref[1]
        den = dp_ref[0] + dp_ref[1]
        rden = 1.0 / (den + 1e-16)
        xx = acc * rden[:, None] + b_ref[...][None, :]
        xx = jnp.where(xx > 0.0, xx, jnp.exp(xx) - 1.0)
        h2 = jnp.dot(xx, w_ref[...], preferred_element_type=jnp.float32)
        h_ref[...] = h2
        sd_ref[...] = jnp.dot(h2, a_ref[...], preferred_element_type=jnp.float32)

    return pl.pallas_call(
        body,
        out_shape=[
            jax.ShapeDtypeStruct((NP, 32), jnp.float32),
            jax.ShapeDtypeStruct((NP, 2), jnp.float32),
        ],
    )(op1, dp1, b1, W2, A2)


def _tc_back(op2, dp2, b2):
    def body(op_ref, dp_ref, b_ref, o_ref):
        acc = op_ref[0] + op_ref[1]
        den = dp_ref[0] + dp_ref[1]
        rden = 1.0 / (den + 1e-16)
        o_ref[...] = acc * rden[:, None] + b_ref[...][None, :]

    return pl.pallas_call(
        body,
        out_shape=jax.ShapeDtypeStruct((NP, 32), jnp.float32),
    )(op2, dp2, b2)


def kernel(x, edge_index, W1, a1_src, a1_dst, b1, W2, a2_src, a2_dst, b2):
    src = edge_index[0].reshape(NW, EPW)
    dst = edge_index[1].reshape(NW, EPW)
    src = jnp.pad(src, ((0, 0), (0, EPWP - EPW)), constant_values=N)
    dst = jnp.pad(dst, ((0, 0), (0, EPWP - EPW)), constant_values=N)
    src = src.reshape(NW, NCHP, CP)
    dst = dst.reshape(NW, NCHP, CP)
    A1 = jnp.stack([a1_src, a1_dst], axis=1)
    A2 = jnp.stack([a2_src, a2_dst], axis=1)
    xp = jnp.pad(x, ((0, NP - N), (0, 0)))

    h1, sd1 = _tc_front(xp, W1, A1)
    dp1, op1 = _sc_layer1(src, dst, sd1[:, 0], sd1[:, 1], h1)
    h2, sd2 = _tc_mid(op1, dp1, b1, W2, A2)
    dp2, op2 = _sc_layer2(src, dst, sd2[:, 0], sd2[:, 1], h2)
    out = _tc_back(op2, dp2, b2)
    return out[:N]


# speedup vs baseline: 43.7749x; 1.2568x over previous
import functools

import jax
import jax.numpy as jnp
from jax import lax
from jax.experimental import pallas as pl
from jax.experimental.pallas import tpu as pltpu
from jax.experimental.pallas import tpu_sc as plsc

N = 10000
E = 320000
NC, NS = 2, 16
NW = NC * NS
EPW = E // NW
CP = 128
EPWP = 10240
NCHP = EPWP // CP
NP = 10240
RPT = NP // NS
NBUF = 4


def _sc_layer(D):
    D2 = D // 2
    mesh = plsc.VectorSubcoreMesh(core_axis_name="c", subcore_axis_name="s")

    @functools.partial(
        pl.kernel,
        out_type=[
            jax.ShapeDtypeStruct((NC, NP), jnp.float32),
            jax.ShapeDtypeStruct((NC, NP, D2), jnp.float32),
        ],
        mesh=mesh,
        compiler_params=pltpu.CompilerParams(
            needs_layout_passes=False, use_tc_tiling_on_sc=False),
        scratch_types=[
            pltpu.VMEM((NCHP, CP), jnp.int32),
            pltpu.VMEM((NCHP, CP), jnp.int32),
            pltpu.VMEM((NP,), jnp.float32),
            pltpu.VMEM((NP,), jnp.float32),
            pltpu.VMEM((EPWP,), jnp.float32),
            pltpu.VMEM((NBUF, CP, D2), jnp.float32),
            pltpu.VMEM((NBUF, CP, D2), jnp.float32),
            pltpu.VMEM((RPT,), jnp.float32),
            pltpu.VMEM_SHARED((NP,), jnp.float32),
            pltpu.VMEM_SHARED((NP, D2), jnp.float32),
            pltpu.VMEM_SHARED((NP, D2), jnp.float32),
            pltpu.SemaphoreType.DMA((NBUF,)),
            pltpu.SemaphoreType.DMA((NBUF,)),
            pltpu.SemaphoreType.DMA((NBUF,)),
            pltpu.SemaphoreType.DMA,
        ],
    )
    def k(src_hbm, dst_hbm, s_hbm, d_hbm, h_hbm, den_out, out_out,
          srcv, dstv, sv, dv, exv, rows, rows_out, zrow, dacc, oacc, hbuf,
          gsem, ssem, dsem, hsem):
        cid = lax.axis_index("c")
        sid = lax.axis_index("s")

        zero = jnp.zeros((16,), jnp.float32)

        pltpu.async_copy(s_hbm, sv, gsem.at[2])
        pltpu.async_copy(d_hbm, dv, gsem.at[3])
        pltpu.async_copy(h_hbm.at[cid, pl.ds(sid * RPT, RPT)],
                         hbuf.at[pl.ds(sid * RPT, RPT)], hsem)

        @pl.loop(0, RPT // 16, unroll=8)
        def _(i):
            zrow[pl.ds(i * 16, 16)] = zero

        @pl.loop(0, CP, unroll=4)
        def _(r):
            for t in range(D2 // 16):
                rows_out[0, r, pl.ds(t * 16, 16)] = zero

        pltpu.sync_copy(zrow, dacc.at[pl.ds(sid * RPT, RPT)])
        for t in range(RPT // CP):
            pltpu.sync_copy(rows_out.at[0],
                            oacc.at[pl.ds(sid * RPT + t * CP, CP)])

        pltpu.make_async_copy(s_hbm, sv, gsem.at[2]).wait()
        pltpu.make_async_copy(d_hbm, dv, gsem.at[3]).wait()
        pltpu.make_async_copy(h_hbm.at[cid, pl.ds(sid * RPT, RPT)],
                              hbuf.at[pl.ds(sid * RPT, RPT)], hsem).wait()
        plsc.subcore_barrier()

        for p in range(2):
            wp = sid * 2 + p
            pltpu.async_copy(src_hbm.at[wp], srcv, gsem.at[0])
            pltpu.async_copy(dst_hbm.at[wp], dstv, gsem.at[1])
            pltpu.make_async_copy(src_hbm.at[wp], srcv, gsem.at[0]).wait()
            pltpu.make_async_copy(dst_hbm.at[wp], dstv, gsem.at[1]).wait()

            @plsc.parallel_loop(0, EPWP // 16, unroll=8)
            def _(v):
                row = v >> 3
                col = (v & 7) * 16
                si = srcv[row, pl.ds(col, 16)]
                di = dstv[row, pl.ds(col, 16)]
                e = plsc.load_gather(sv, [si]) + plsc.load_gather(dv, [di])
                e = jnp.where(e >= 0.0, e, 0.2 * e)
                exv[pl.ds(v * 16, 16)] = jnp.exp(e)

            for c in range(2):
                pltpu.async_copy(hbuf.at[srcv.at[c]], rows.at[c], gsem.at[c])

            @pl.loop(0, NCHP, step=NBUF)
            def _(cbase):
                for b in range(NBUF):
                    c = cbase + b
                    bn = (b + 2) % NBUF

                    @pl.when(c < NCHP - 2)
                    def _():
                        pltpu.async_copy(
                            hbuf.at[srcv.at[c + 2]], rows.at[bn],
                            gsem.at[bn])

                    pltpu.make_async_copy(
                        hbuf.at[srcv.at[c]], rows.at[b], gsem.at[b]).wait()

                    @pl.when(c >= NBUF)
                    def _():
                        pltpu.make_async_copy(
                            rows_out.at[b], oacc.at[dstv.at[c - NBUF]],
                            ssem.at[b]).wait()

                        @pl.when(cid == 0)
                        def _():
                            pltpu.make_async_copy(
                                exv.at[pl.ds((c - NBUF) * CP, CP)],
                                dacc.at[dstv.at[c - NBUF]],
                                dsem.at[b]).wait()

                    @plsc.parallel_loop(0, CP, unroll=8)
                    def _(r):
                        aidx = jnp.full((16,), c * CP + r, jnp.int32)
                        a = plsc.load_gather(exv, [aidx])
                        for t in range(D2 // 16):
                            rows_out[b, r, pl.ds(t * 16, 16)] = (
                                rows[b, r, pl.ds(t * 16, 16)] * a)

                    pltpu.async_copy(
                        rows_out.at[b], oacc.at[dstv.at[c]], ssem.at[b],
                        add=True)

                    @pl.when(cid == 0)
                    def _():
                        pltpu.async_copy(
                            exv.at[pl.ds(c * CP, CP)], dacc.at[dstv.at[c]],
                            dsem.at[b], add=True)

            for c in range(NCHP - NBUF, NCHP):
                b = c % NBUF
                pltpu.make_async_copy(
                    rows_out.at[b], oacc.at[dstv.at[c]], ssem.at[b]).wait()

                @pl.when(cid == 0)
                def _():
                    pltpu.make_async_copy(
                        exv.at[pl.ds(c * CP, CP)], dacc.at[dstv.at[c]],
                        dsem.at[b]).wait()

        plsc.subcore_barrier()

        pltpu.sync_copy(dacc.at[pl.ds(sid * RPT, RPT)],
                        den_out.at[cid, pl.ds(sid * RPT, RPT)])
        pltpu.sync_copy(oacc.at[pl.ds(sid * RPT, RPT)],
                        out_out.at[cid, pl.ds(sid * RPT, RPT)])

    return k


_sc_layer1 = _sc_layer(64)
_sc_layer2 = _sc_layer(32)


def _tc_front(x, W1, A1):
    def body(x_ref, w_ref, a_ref, h_ref, sd_ref):
        h = jnp.dot(x_ref[...], w_ref[...], preferred_element_type=jnp.float32)
        h_ref[0] = h[:, :32]
        h_ref[1] = h[:, 32:]
        sd_ref[...] = jnp.dot(h, a_ref[...], preferred_element_type=jnp.float32)

    return pl.pallas_call(
        body,
        out_shape=[
            jax.ShapeDtypeStruct((NC, NP, 32), jnp.float32),
            jax.ShapeDtypeStruct((NP, 2), jnp.float32),
        ],
    )(x, W1, A1)


def _tc_mid(op1, dp1, b1, W2, A2):
    def body(op_ref, dp_ref, b_ref, w_ref, a_ref, h_ref, sd_ref):
        acc = jnp.concatenate([op_ref[0], op_ref[1]], axis=1)
        den = dp_ref[0] + dp_ref[1]
        rden = 1.0 / (den + 1e-16)
        xx = acc * rden[:, None] + b_ref[...][None, :]
        xx = jnp.where(xx > 0.0, xx, jnp.exp(xx) - 1.0)
        h2 = jnp.dot(xx, w_ref[...], preferred_element_type=jnp.float32)
        h_ref[0] = h2[:, :16]
        h_ref[1] = h2[:, 16:]
        sd_ref[...] = jnp.dot(h2, a_ref[...], preferred_element_type=jnp.float32)

    return pl.pallas_call(
        body,
        out_shape=[
            jax.ShapeDtypeStruct((NC, NP, 16), jnp.float32),
            jax.ShapeDtypeStruct((NP, 2), jnp.float32),
        ],
    )(op1, dp1, b1, W2, A2)


def _tc_back(op2, dp2, b2):
    def body(op_ref, dp_ref, b_ref, o_ref):
        acc = jnp.concatenate([op_ref[0], op_ref[1]], axis=1)
        den = dp_ref[0] + dp_ref[1]
        rden = 1.0 / (den + 1e-16)
        o_ref[...] = acc * rden[:, None] + b_ref[...][None, :]

    return pl.pallas_call(
        body,
        out_shape=jax.ShapeDtypeStruct((NP, 32), jnp.float32),
    )(op2, dp2, b2)


def kernel(x, edge_index, W1, a1_src, a1_dst, b1, W2, a2_src, a2_dst, b2):
    src = edge_index[0].reshape(NW, EPW)
    dst = edge_index[1].reshape(NW, EPW)
    src = jnp.pad(src, ((0, 0), (0, EPWP - EPW)), constant_values=N)
    dst = jnp.pad(dst, ((0, 0), (0, EPWP - EPW)), constant_values=N)
    src = src.reshape(NW, NCHP, CP)
    dst = dst.reshape(NW, NCHP, CP)
    A1 = jnp.stack([a1_src, a1_dst], axis=1)
    A2 = jnp.stack([a2_src, a2_dst], axis=1)
    xp = jnp.pad(x, ((0, NP - N), (0, 0)))

    h1, sd1 = _tc_front(xp, W1, A1)
    dp1, op1 = _sc_layer1(src, dst, sd1[:, 0], sd1[:, 1], h1)
    h2, sd2 = _tc_mid(op1, dp1, b1, W2, A2)
    dp2, op2 = _sc_layer2(src, dst, sd2[:, 0], sd2[:, 1], h2)
    out = _tc_back(op2, dp2, b2)
    return out[:N]
